# baseline (speedup 1.0000x reference)
import jax
import jax.numpy as jnp
from jax.experimental import pallas as pl
from jax.experimental.pallas import tpu as pltpu

_BLOCK = 20000
_SUB = 5000
_NSUB = _BLOCK // _SUB


def _body(x_ref, pe_ref, w1_ref, w2_ref, b1_ref, b2_ref, coff_ref, t_ref,
          o_hbm, pe_s, w2_s, obuf, osems):
    d = w2_ref.shape[0]
    i = pl.program_id(0)

    @pl.when(i == 0)
    def _prep():
        trow = t_ref[0, 0] + 1
        pe_s[...] = pe_ref[pl.ds(trow, 1), :] * coff_ref[0, 0]
        w2_s[...] = jnp.broadcast_to(w2_ref[:, 1:2], (d, d))

    for s in range(_NSUB):
        def out_copy(row0, slot):
            return pltpu.make_async_copy(
                obuf.at[slot], o_hbm.at[pl.ds(row0, _SUB), :], osems.at[slot])

        @pl.when(i > 0)
        def _retire():
            out_copy((i - 1) * _BLOCK + s * _SUB, s).wait()

        x = x_ref[pl.ds(s * _SUB, _SUB), :] + pe_s[...]
        h = jnp.dot(x, w1_ref[...], preferred_element_type=jnp.float32)
        h = jnp.maximum(h + b1_ref[...], 0.0)
        m = jnp.dot(h, w2_s[...], preferred_element_type=jnp.float32)
        m = jnp.maximum(m + b2_ref[0, 1], 0.0)
        obuf[s] = x * jax.nn.sigmoid(m)
        out_copy(i * _BLOCK + s * _SUB, s).start()

    @pl.when(i == pl.num_programs(0) - 1)
    def _drain():
        for s in range(_NSUB):
            pltpu.make_async_copy(
                obuf.at[s], o_hbm.at[pl.ds(s * _SUB, _SUB), :],
                osems.at[s]).wait()


def kernel(graph, input, W1, b1, W2, b2, pe_coff, pe, t):
    n, d = input.shape
    assert n % _BLOCK == 0
    grid = (n // _BLOCK,)

    vmem = pl.BlockSpec(memory_space=pltpu.VMEM)
    smem = pl.BlockSpec(memory_space=pltpu.SMEM)
    return pl.pallas_call(
        _body,
        grid=grid,
        in_specs=[
            pl.BlockSpec((_BLOCK, d), lambda i: (i, 0)),
            vmem, vmem, vmem, vmem,
            smem, smem, smem,
        ],
        out_specs=pl.BlockSpec(memory_space=pl.ANY),
        out_shape=jax.ShapeDtypeStruct((n, d), jnp.float32),
        scratch_shapes=[
            pltpu.VMEM((1, d), jnp.float32),
            pltpu.VMEM((d, d), jnp.float32),
            pltpu.VMEM((_NSUB, _SUB, d), jnp.float32),
            pltpu.SemaphoreType.DMA((_NSUB,)),
        ],
        compiler_params=pltpu.CompilerParams(
            dimension_semantics=("arbitrary",),
        ),
    )(
        input, pe, W1, W2,
        b1.reshape(1, d),
        b2.reshape(1, 2),
        pe_coff.reshape(1, 1),
        jnp.asarray(t, jnp.int32).reshape(1, 1),
    )


# speedup vs baseline: 1.1072x; 1.1072x over previous
import jax
import jax.numpy as jnp
from jax.experimental import pallas as pl
from jax.experimental.pallas import tpu as pltpu

_BLOCK = 20000
_SUB = 4000
_NSUB = _BLOCK // _SUB


def _body(x_ref, pe_ref, w1_ref, w2_ref, b1_ref, b2_ref, coff_ref, t_ref,
          o_hbm, pe_s, w2_s, obuf, osems):
    d = w2_ref.shape[0]
    i = pl.program_id(0)

    @pl.when(i == 0)
    def _prep():
        trow = t_ref[0, 0] + 1
        pe_s[...] = pe_ref[pl.ds(trow, 1), :] * coff_ref[0, 0]
        w2_s[...] = jnp.broadcast_to(w2_ref[:, 1:2], (d, d))

    bank = (i % 2) * _NSUB
    for s in range(_NSUB):
        def out_copy(row0, slot):
            return pltpu.make_async_copy(
                obuf.at[slot], o_hbm.at[pl.ds(row0, _SUB), :], osems.at[slot])

        @pl.when(i > 1)
        def _retire():
            out_copy((i - 2) * _BLOCK + s * _SUB, bank + s).wait()

        x = x_ref[pl.ds(s * _SUB, _SUB), :] + pe_s[...]
        h = jnp.dot(x, w1_ref[...], preferred_element_type=jnp.float32)
        h = jnp.maximum(h + b1_ref[...], 0.0)
        m = jnp.dot(h, w2_s[...], preferred_element_type=jnp.float32)
        m = jnp.maximum(m + b2_ref[0, 1], 0.0)
        obuf[bank + s] = x * jax.nn.sigmoid(m)
        out_copy(i * _BLOCK + s * _SUB, bank + s).start()

    ng = pl.num_programs(0)

    @pl.when(i == ng - 1)
    def _drain():
        for s in range(_NSUB):
            @pl.when(ng > 1)
            def _prev():
                pltpu.make_async_copy(
                    obuf.at[((ng - 2) % 2) * _NSUB + s],
                    o_hbm.at[pl.ds(s * _SUB, _SUB), :],
                    osems.at[((ng - 2) % 2) * _NSUB + s]).wait()
            pltpu.make_async_copy(
                obuf.at[bank + s], o_hbm.at[pl.ds(s * _SUB, _SUB), :],
                osems.at[bank + s]).wait()


def kernel(graph, input, W1, b1, W2, b2, pe_coff, pe, t):
    n, d = input.shape
    assert n % _BLOCK == 0
    grid = (n // _BLOCK,)

    vmem = pl.BlockSpec(memory_space=pltpu.VMEM)
    smem = pl.BlockSpec(memory_space=pltpu.SMEM)
    return pl.pallas_call(
        _body,
        grid=grid,
        in_specs=[
            pl.BlockSpec((_BLOCK, d), lambda i: (i, 0)),
            vmem, vmem, vmem, vmem,
            smem, smem, smem,
        ],
        out_specs=pl.BlockSpec(memory_space=pl.ANY),
        out_shape=jax.ShapeDtypeStruct((n, d), jnp.float32),
        scratch_shapes=[
            pltpu.VMEM((1, d), jnp.float32),
            pltpu.VMEM((d, d), jnp.float32),
            pltpu.VMEM((2 * _NSUB, _SUB, d), jnp.float32),
            pltpu.SemaphoreType.DMA((2 * _NSUB,)),
        ],
        compiler_params=pltpu.CompilerParams(
            dimension_semantics=("arbitrary",),
        ),
    )(
        input, pe, W1, W2,
        b1.reshape(1, d),
        b2.reshape(1, 2),
        pe_coff.reshape(1, 1),
        jnp.asarray(t, jnp.int32).reshape(1, 1),
    )


# speedup vs baseline: 1.1136x; 1.0058x over previous
import jax
import jax.numpy as jnp
from jax.experimental import pallas as pl
from jax.experimental.pallas import tpu as pltpu

_BLOCK = 20000
_SUB = 4000
_NSUB = _BLOCK // _SUB


def _body(x_ref, pe_ref, w1_ref, w2_ref, b1_ref, b2_ref, coff_ref, t_ref,
          o_hbm, pe_s, w2_s, obuf, osems):
    d = w2_ref.shape[0]
    i = pl.program_id(0)

    @pl.when(i == 0)
    def _prep():
        trow = t_ref[0, 0] + 1
        pe_s[...] = pe_ref[pl.ds(trow, 1), :] * coff_ref[0, 0]
        w2_s[...] = jnp.broadcast_to(w2_ref[:, 1:2], (d, d))

    for s in range(_NSUB):
        def out_copy(row0, slot):
            return pltpu.make_async_copy(
                obuf.at[slot], o_hbm.at[pl.ds(row0, _SUB), :], osems.at[slot])

        @pl.when(i > 0)
        def _retire():
            out_copy((i - 1) * _BLOCK + s * _SUB, s).wait()

        x = x_ref[pl.ds(s * _SUB, _SUB), :] + pe_s[...]
        h = jnp.dot(x, w1_ref[...], preferred_element_type=jnp.float32)
        h = jnp.maximum(h + b1_ref[...], 0.0)
        m = jnp.dot(h, w2_s[...], preferred_element_type=jnp.float32)
        m = jnp.maximum(m + b2_ref[0, 1], 0.0)
        obuf[s] = x * jax.nn.sigmoid(m)
        out_copy(i * _BLOCK + s * _SUB, s).start()

    @pl.when(i == pl.num_programs(0) - 1)
    def _drain():
        for s in range(_NSUB):
            pltpu.make_async_copy(
                obuf.at[s], o_hbm.at[pl.ds(s * _SUB, _SUB), :],
                osems.at[s]).wait()


def kernel(graph, input, W1, b1, W2, b2, pe_coff, pe, t):
    n, d = input.shape
    assert n % _BLOCK == 0
    grid = (n // _BLOCK,)

    vmem = pl.BlockSpec(memory_space=pltpu.VMEM)
    smem = pl.BlockSpec(memory_space=pltpu.SMEM)
    return pl.pallas_call(
        _body,
        grid=grid,
        in_specs=[
            pl.BlockSpec((_BLOCK, d), lambda i: (i, 0)),
            vmem, vmem, vmem, vmem,
            smem, smem, smem,
        ],
        out_specs=pl.BlockSpec(memory_space=pl.ANY),
        out_shape=jax.ShapeDtypeStruct((n, d), jnp.float32),
        scratch_shapes=[
            pltpu.VMEM((1, d), jnp.float32),
            pltpu.VMEM((d, d), jnp.float32),
            pltpu.VMEM((_NSUB, _SUB, d), jnp.float32),
            pltpu.SemaphoreType.DMA((_NSUB,)),
        ],
        compiler_params=pltpu.CompilerParams(
            dimension_semantics=("arbitrary",),
        ),
    )(
        input, pe, W1, W2,
        b1.reshape(1, d),
        b2.reshape(1, 2),
        pe_coff.reshape(1, 1),
        jnp.asarray(t, jnp.int32).reshape(1, 1),
    )
